# Initial kernel scaffold; baseline (speedup 1.0000x reference)
import jax
import jax.numpy as jnp
from jax.experimental import pallas as pl


def kernel(x, edge_index, batch_node_id, W1_0, b1_0, gamma_0, beta_0, W2_0, b2_0, W1_1, b1_1, gamma_1, beta_1, W2_1, b2_1):
    raise NotImplementedError("write your pallas kernel here")


# speedup vs baseline: 5.3889x; 5.3889x over previous
import functools

import jax
import jax.numpy as jnp
from jax import lax
from jax.experimental import pallas as pl
from jax.experimental.pallas import tpu as pltpu
from jax.experimental.pallas import tpu_sc as plsc

N = 10000
E = 320000
D = 128
G = 8

NC = 2
NS = 16
NW = NC * NS
EPW = E // NW
CH = 80
NCHUNK = EPW // CH
RPT = 624
TAIL = N - NS * RPT
ZROWS = 48

_mesh = plsc.VectorSubcoreMesh(core_axis_name="c", subcore_axis_name="s")


@functools.partial(
    pl.kernel,
    out_type=jax.ShapeDtypeStruct((NC, N, D), jnp.float32),
    mesh=_mesh,
    scratch_types=[
        pltpu.VMEM((CH,), jnp.int32),
        pltpu.VMEM((CH,), jnp.int32),
        pltpu.VMEM((CH, D), jnp.float32),
        pltpu.VMEM((ZROWS, D), jnp.float32),
        pltpu.VMEM_SHARED((N, D), jnp.float32),
        pltpu.SemaphoreType.DMA,
    ],
)
def _sc_agg(h_hbm, src_hbm, dst_hbm, out_hbm, src_v, dst_v, rows_v, zbuf, acc_sh, sem):
    c = lax.axis_index("c")
    s = lax.axis_index("s")
    wid = c * NS + s

    zv = jnp.zeros((16,), jnp.float32)

    def _zrow(i, _):
        def _zcol(j, _):
            zbuf[i, pl.ds(j * 16, 16)] = zv
            return 0
        return lax.fori_loop(0, D // 16, _zcol, 0)

    lax.fori_loop(0, ZROWS, _zrow, 0)

    def _zcp(k, _):
        pltpu.sync_copy(zbuf, acc_sh.at[pl.ds(s * RPT + k * ZROWS, ZROWS)])
        return 0

    lax.fori_loop(0, RPT // ZROWS, _zcp, 0)

    @pl.when(s == NS - 1)
    def _ztail():
        pltpu.sync_copy(zbuf.at[pl.ds(0, TAIL)], acc_sh.at[pl.ds(NS * RPT, TAIL)])

    plsc.subcore_barrier()

    base = wid * EPW

    def _chunk(i, _):
        off = base + i * CH
        pltpu.sync_copy(src_hbm.at[pl.ds(off, CH)], src_v)
        pltpu.sync_copy(dst_hbm.at[pl.ds(off, CH)], dst_v)
        pltpu.async_copy(h_hbm.at[src_v], rows_v, sem).wait()
        pltpu.sync_copy(rows_v, acc_sh.at[dst_v], add=True)
        return 0

    lax.fori_loop(0, NCHUNK, _chunk, 0)
    plsc.subcore_barrier()

    pltpu.sync_copy(acc_sh.at[pl.ds(s * RPT, RPT)],
                    out_hbm.at[c, pl.ds(s * RPT, RPT)])

    @pl.when(s == NS - 1)
    def _wtail():
        pltpu.sync_copy(acc_sh.at[pl.ds(NS * RPT, TAIL)],
                        out_hbm.at[c, pl.ds(NS * RPT, TAIL)])


def _mlp_pool_body(emit_h, h_ref, agg_ref, batch_ref,
                   W1_ref, b1_ref, g_ref, be_ref, W2_ref, b2_ref, *outs):
    z = h_ref[...] + agg_ref[0] + agg_ref[1]
    z = jnp.dot(z, W1_ref[...], preferred_element_type=jnp.float32) + b1_ref[...]
    mean = jnp.mean(z, axis=0, keepdims=True)
    var = jnp.mean(z * z, axis=0, keepdims=True) - mean * mean
    z = (z - mean) * (g_ref[...] * lax.rsqrt(var + 1e-5)) + be_ref[...]
    z = jnp.maximum(z, 0.0)
    z = jnp.dot(z, W2_ref[...], preferred_element_type=jnp.float32) + b2_ref[...]
    h = jnp.maximum(z, 0.0)
    onehot = (batch_ref[...] ==
              lax.broadcasted_iota(jnp.int32, (G, N), 0)).astype(jnp.float32)
    pool = jnp.dot(onehot, h, preferred_element_type=jnp.float32)
    if emit_h:
        outs[0][...] = h
        outs[1][...] = pool
    else:
        outs[0][...] = pool


def _tc_layer(h, agg2, batch2d, W1, b1, gamma, beta, W2, b2, emit_h):
    if emit_h:
        out_shape = (jax.ShapeDtypeStruct((N, D), jnp.float32),
                     jax.ShapeDtypeStruct((G, D), jnp.float32))
    else:
        out_shape = (jax.ShapeDtypeStruct((G, D), jnp.float32),)
    return pl.pallas_call(
        functools.partial(_mlp_pool_body, emit_h),
        out_shape=out_shape,
    )(h, agg2, batch2d,
      W1, b1.reshape(1, D), gamma.reshape(1, D), beta.reshape(1, D),
      W2, b2.reshape(1, D))


def kernel(x, edge_index, batch_node_id,
           W1_0, b1_0, gamma_0, beta_0, W2_0, b2_0,
           W1_1, b1_1, gamma_1, beta_1, W2_1, b2_1):
    src = edge_index[0]
    dst = edge_index[1]
    batch2d = batch_node_id.reshape(1, N)

    agg_x = _sc_agg(x, src, dst)
    h1, pool1 = _tc_layer(x, agg_x, batch2d,
                          W1_0, b1_0, gamma_0, beta_0, W2_0, b2_0, True)
    agg_h1 = _sc_agg(h1, src, dst)
    (pool2,) = _tc_layer(h1, agg_h1, batch2d,
                         W1_1, b1_1, gamma_1, beta_1, W2_1, b2_1, False)
    return jnp.concatenate([pool1, pool2], axis=1)
